# Initial kernel scaffold; baseline (speedup 1.0000x reference)
import jax
import jax.numpy as jnp
from jax.experimental import pallas as pl


def kernel(x, time_day, time_week):
    raise NotImplementedError("write your pallas kernel here")


# speedup vs baseline: 1.6422x; 1.6422x over previous
import functools

import jax
import jax.numpy as jnp
from jax import lax
from jax.experimental import pallas as pl
from jax.experimental.pallas import tpu as pltpu
from jax.experimental.pallas import tpu_sc as plsc

TIME = 288
WEEK = 7
B, T, N, C = 64, 12, 4096, 3
F = 128

NC, NS, L = 2, 16, 16
NW = NC * NS
B_PER_W = B // NW
FBLK = 8
NCHUNK = N // L


def _sc_body(x_hbm, day_hbm, week_hbm, out_hbm,
             xbuf, didx, widx, day_v, week_v, obuf):
    wid = lax.axis_index("s") * NC + lax.axis_index("c")
    pltpu.sync_copy(day_hbm, day_v)
    pltpu.sync_copy(week_hbm, week_v)
    iota = lax.iota(jnp.int32, L)

    for b_local in range(B_PER_W):
        b = wid * B_PER_W + b_local
        pltpu.sync_copy(x_hbm.at[b, pl.ds((T - 1) * N * C, N * C)], xbuf)

        @pl.loop(0, NCHUNK)
        def _idx_pass(i):
            g = iota * C + i * (L * C)
            xd = plsc.load_gather(xbuf, [g + 1])
            xw = plsc.load_gather(xbuf, [g + 2])
            d = jnp.clip((xd * TIME).astype(jnp.int32), 0, TIME - 1)
            w = jnp.clip(xw.astype(jnp.int32), 0, WEEK - 1)
            didx[pl.ds(i * L, L)] = d
            widx[pl.ds(i * L, L)] = w

        @pl.loop(0, F // FBLK)
        def _f_block(fb):
            f0 = fb * FBLK

            @pl.loop(0, NCHUNK)
            def _chunk(i):
                d = didx[pl.ds(i * L, L)]
                w = widx[pl.ds(i * L, L)]
                for fl in range(FBLK):
                    f = jnp.full((L,), f0 + fl, jnp.int32)
                    vd = plsc.load_gather(day_v, [d, f])
                    vw = plsc.load_gather(week_v, [w, f])
                    obuf[fl, pl.ds(i * L, L)] = vd + vw

            pltpu.sync_copy(obuf, out_hbm.at[b, pl.ds(f0, FBLK), :])


@jax.jit
def _temporal_embedding(x2d, time_day, time_week):
    run = pl.kernel(
        _sc_body,
        out_type=jax.ShapeDtypeStruct((B, F, N), jnp.float32),
        mesh=plsc.VectorSubcoreMesh(core_axis_name="c", subcore_axis_name="s"),
        scratch_types=[
            pltpu.VMEM((N * C,), jnp.float32),
            pltpu.VMEM((N,), jnp.int32),
            pltpu.VMEM((N,), jnp.int32),
            pltpu.VMEM((TIME, F), jnp.float32),
            pltpu.VMEM((WEEK, F), jnp.float32),
            pltpu.VMEM((FBLK, N), jnp.float32),
        ],
        compiler_params=pltpu.CompilerParams(needs_layout_passes=False),
    )
    return run(x2d, time_day, time_week)


def kernel(x, time_day, time_week):
    x2d = x.reshape(B, T * N * C)
    out = _temporal_embedding(x2d, time_day, time_week)
    return out[..., None]
